# baseline (speedup 1.0000x reference)
import functools
import jax
import jax.numpy as jnp
from jax import lax
from jax.experimental import pallas as pl
from jax.experimental.pallas import tpu as pltpu
from jax.experimental.pallas import tpu_sc as plsc

D = 64
SCALE = 8.0
NC = 2
NS = 16
NW = NC * NS
L = 16

B_TOTAL = 4096 * 200
B_PER_W = B_TOTAL // NW
CHUNK = 512
N_CHUNKS = B_PER_W // CHUNK
ROW_VECS = D // L


def _emb_body(x_hbm, lut_hbm, out_hbm, idx_v, rows_v, sem):
    wid = lax.axis_index("s") * NC + lax.axis_index("c")
    base = wid * B_PER_W

    def chunk_body(g, carry):
        off = base + g * CHUNK
        pltpu.sync_copy(x_hbm.at[pl.ds(off, CHUNK)], idx_v)
        pltpu.async_copy(lut_hbm.at[idx_v], rows_v, sem).wait()

        def scale_body(r, c2):
            for u in range(4):
                row = r * 4 + u
                for j in range(ROW_VECS):
                    sl = pl.ds(j * L, L)
                    rows_v[row, sl] = rows_v[row, sl] * SCALE
            return c2

        lax.fori_loop(0, CHUNK // 4, scale_body, 0)
        pltpu.sync_copy(rows_v, out_hbm.at[pl.ds(off, CHUNK)])
        return carry

    lax.fori_loop(0, N_CHUNKS, chunk_body, 0)


@jax.jit
def kernel(x, lut):
    xf = x.reshape(-1).astype(jnp.int32)
    call = pl.kernel(
        _emb_body,
        out_type=jax.ShapeDtypeStruct((B_TOTAL, D), jnp.float32),
        mesh=plsc.VectorSubcoreMesh(core_axis_name="c", subcore_axis_name="s"),
        scratch_types=[
            pltpu.VMEM((CHUNK,), jnp.int32),
            pltpu.VMEM((CHUNK, D), jnp.float32),
            pltpu.SemaphoreType.DMA,
        ],
        compiler_params=pltpu.CompilerParams(use_tc_tiling_on_sc=False),
    )
    out = call(xf, lut)
    return out.reshape(x.shape[0], x.shape[1], D)


# speedup vs baseline: 1.0918x; 1.0918x over previous
import jax
import jax.numpy as jnp
from jax import lax
from jax.experimental import pallas as pl
from jax.experimental.pallas import tpu as pltpu
from jax.experimental.pallas import tpu_sc as plsc

D = 64
SCALE = 8.0
NC = 2
NS = 16
NW = NC * NS
L = 16

B_TOTAL = 4096 * 200
B_PER_W = B_TOTAL // NW
CHUNK = 400
N_CHUNKS = B_PER_W // CHUNK
N_PAIRS = N_CHUNKS // 2
ROWS_PER_IT = 8
ROW_VECS = D // L


def _emb_body(x_hbm, lut_hbm, out_hbm,
              idx0, idx1, rin0, rin1, rout0, rout1,
              gsem0, gsem1, osem0, osem1):
    wid = lax.axis_index("s") * NC + lax.axis_index("c")
    base = wid * B_PER_W
    idx = (idx0, idx1)
    rin = (rin0, rin1)
    rout = (rout0, rout1)
    gsem = (gsem0, gsem1)
    osem = (osem0, osem1)

    def start_gather(g, b):
        off = base + g * CHUNK
        pltpu.sync_copy(x_hbm.at[pl.ds(off, CHUNK)], idx[b])
        pltpu.async_copy(lut_hbm.at[idx[b]], rin[b], gsem[b])

    def wait_gather(b):
        pltpu.make_async_copy(lut_hbm.at[idx[b]], rin[b], gsem[b]).wait()

    def start_out(g, b):
        off = base + g * CHUNK
        pltpu.async_copy(rout[b], out_hbm.at[pl.ds(off, CHUNK)], osem[b])

    def wait_out(b):
        pltpu.make_async_copy(
            rout[b], out_hbm.at[pl.ds(base, CHUNK)], osem[b]).wait()

    def scale(b):
        def body(r, c):
            for u in range(ROWS_PER_IT):
                row = r * ROWS_PER_IT + u
                for j in range(ROW_VECS):
                    sl = pl.ds(j * L, L)
                    rout[b][row, sl] = rin[b][row, sl] * SCALE
            return c
        lax.fori_loop(0, CHUNK // ROWS_PER_IT, body, 0)

    def phase(g, b, first, last):
        if not last:
            start_gather(g + 1, 1 - b)
        wait_gather(b)
        if not first:
            wait_out(b)
        scale(b)
        start_out(g, b)

    start_gather(0, 0)
    phase(0, 0, True, False)
    phase(1, 1, True, False)

    def pair(i, c):
        g = i * 2
        phase(g, 0, False, False)
        phase(g + 1, 1, False, False)
        return c

    lax.fori_loop(1, N_PAIRS - 1, pair, 0)
    phase(N_CHUNKS - 2, 0, False, False)
    phase(N_CHUNKS - 1, 1, False, True)
    wait_out(0)
    wait_out(1)


@jax.jit
def kernel(x, lut):
    xf = x.reshape(-1).astype(jnp.int32)
    call = pl.kernel(
        _emb_body,
        out_type=jax.ShapeDtypeStruct((B_TOTAL, D), jnp.float32),
        mesh=plsc.VectorSubcoreMesh(core_axis_name="c", subcore_axis_name="s"),
        scratch_types=[
            pltpu.VMEM((CHUNK,), jnp.int32),
            pltpu.VMEM((CHUNK,), jnp.int32),
            pltpu.VMEM((CHUNK, D), jnp.float32),
            pltpu.VMEM((CHUNK, D), jnp.float32),
            pltpu.VMEM((CHUNK, D), jnp.float32),
            pltpu.VMEM((CHUNK, D), jnp.float32),
            pltpu.SemaphoreType.DMA,
            pltpu.SemaphoreType.DMA,
            pltpu.SemaphoreType.DMA,
            pltpu.SemaphoreType.DMA,
        ],
        compiler_params=pltpu.CompilerParams(use_tc_tiling_on_sc=False),
    )
    out = call(xf, lut)
    return out.reshape(x.shape[0], x.shape[1], D)
